# Initial kernel scaffold; baseline (speedup 1.0000x reference)
import jax
import jax.numpy as jnp
from jax.experimental import pallas as pl


def kernel(rpn_scores, rpn_reg, xyz, mean_size):
    raise NotImplementedError("write your pallas kernel here")


# speedup vs baseline: 64.5945x; 64.5945x over previous
import math

import jax
import jax.numpy as jnp
from jax import lax
from jax.experimental import pallas as pl
from jax.experimental.pallas import tpu as pltpu

_PRE = 2048
_POST = 512
_THRESH = 0.85
_NBIN = 12
_NHEAD = 12
_BIN = 0.5
_SCOPE = 3.0


def _decode_nms_body(reg_ref, xyz_ref, ms_ref, boxes_ref, keep_ref):
    b = reg_ref.shape[0]
    n = reg_ref.shape[2]
    apc = 2.0 * math.pi / _NHEAD

    ms0 = ms_ref[0:1, :]
    ms1 = ms_ref[1:2, :]
    ms2 = ms_ref[2:3, :]

    x1_rows, y1_rows, x2_rows, y2_rows, ar_rows = [], [], [], [], []
    for k in range(b):
        reg = reg_ref[k]
        xyz = xyz_ref[k]
        iot = lax.broadcasted_iota(jnp.int32, (_NBIN, n), 0)

        sx = reg[0:_NBIN, :]
        mx = jnp.max(sx, axis=0, keepdims=True)
        xbin = jnp.min(jnp.where(sx == mx, iot, _NBIN), axis=0, keepdims=True)
        xres = jnp.sum(
            jnp.where(iot == xbin, reg[2 * _NBIN:3 * _NBIN, :], 0.0),
            axis=0, keepdims=True) * _BIN
        posx = xbin.astype(jnp.float32) * _BIN + (_BIN * 0.5 - _SCOPE) + xres
        posx = posx + xyz[0:1, :]

        sz = reg[_NBIN:2 * _NBIN, :]
        mz = jnp.max(sz, axis=0, keepdims=True)
        zbin = jnp.min(jnp.where(sz == mz, iot, _NBIN), axis=0, keepdims=True)
        zres = jnp.sum(
            jnp.where(iot == zbin, reg[3 * _NBIN:4 * _NBIN, :], 0.0),
            axis=0, keepdims=True) * _BIN
        posz = zbin.astype(jnp.float32) * _BIN + (_BIN * 0.5 - _SCOPE) + zres
        posz = posz + xyz[2:3, :]

        posy = xyz[1:2, :] + reg[4 * _NBIN:4 * _NBIN + 1, :]

        hs = 4 * _NBIN + 1
        sr = reg[hs:hs + _NHEAD, :]
        mr = jnp.max(sr, axis=0, keepdims=True)
        rybin = jnp.min(jnp.where(sr == mr, iot, _NHEAD), axis=0, keepdims=True)
        ryres = jnp.sum(
            jnp.where(iot == rybin, reg[hs + _NHEAD:hs + 2 * _NHEAD, :], 0.0),
            axis=0, keepdims=True)
        ry = jnp.mod(rybin.astype(jnp.float32) * apc + ryres * (apc * 0.5),
                     2.0 * math.pi)
        ry = jnp.where(ry > math.pi, ry - 2.0 * math.pi, ry)

        hws = hs + 2 * _NHEAD
        h = reg[hws:hws + 1, :] * ms0 + ms0
        w = reg[hws + 1:hws + 2, :] * ms1 + ms1
        l = reg[hws + 2:hws + 3, :] * ms2 + ms2
        posy = posy + h * 0.5

        boxes_ref[k] = jnp.concatenate(
            [posx, posy, posz, h, w, l, ry, jnp.zeros_like(ry)], axis=0)

        hl = l * 0.5
        hw = w * 0.5
        x1 = posx - hl
        y1 = posz - hw
        x2 = posx + hl
        y2 = posz + hw
        x1_rows.append(x1)
        y1_rows.append(y1)
        x2_rows.append(x2)
        y2_rows.append(y2)
        ar_rows.append((x2 - x1) * (y2 - y1))

    x1m = jnp.concatenate(x1_rows, axis=0)
    y1m = jnp.concatenate(y1_rows, axis=0)
    x2m = jnp.concatenate(x2_rows, axis=0)
    y2m = jnp.concatenate(y2_rows, axis=0)
    arm = jnp.concatenate(ar_rows, axis=0)
    li = lax.broadcasted_iota(jnp.int32, (b, n), 1)

    def body(i, keep_f):
        sel = li == i
        xi = jnp.sum(jnp.where(sel, x1m, 0.0), axis=1, keepdims=True)
        yi = jnp.sum(jnp.where(sel, y1m, 0.0), axis=1, keepdims=True)
        x2i = jnp.sum(jnp.where(sel, x2m, 0.0), axis=1, keepdims=True)
        y2i = jnp.sum(jnp.where(sel, y2m, 0.0), axis=1, keepdims=True)
        ai = jnp.sum(jnp.where(sel, arm, 0.0), axis=1, keepdims=True)
        ki = jnp.max(jnp.where(sel, keep_f, 0.0), axis=1, keepdims=True)
        xx1 = jnp.maximum(x1m, xi)
        yy1 = jnp.maximum(y1m, yi)
        xx2 = jnp.minimum(x2m, x2i)
        yy2 = jnp.minimum(y2m, y2i)
        inter = jnp.maximum(xx2 - xx1, 0.0) * jnp.maximum(yy2 - yy1, 0.0)
        iou = inter / (ai + arm - inter + 1e-8)
        sup = (iou > _THRESH) & (li > i) & (ki > 0.5)
        return jnp.where(sup, 0.0, keep_f)

    keep_ref[:] = lax.fori_loop(0, n, body, jnp.ones((b, n), jnp.float32))


def kernel(rpn_scores, rpn_reg, xyz, mean_size):
    bsz, _ = rpn_scores.shape
    scores, idx = lax.top_k(rpn_scores, _PRE)
    reg_top = jnp.take_along_axis(rpn_reg, idx[:, :, None], axis=1)
    xyz_top = jnp.take_along_axis(xyz, idx[:, :, None], axis=1)
    reg_t = reg_top.transpose(0, 2, 1)
    xyz_t = xyz_top.transpose(0, 2, 1)
    ms_b = jnp.broadcast_to(mean_size[:, None], (3, _PRE))

    boxes8, keep_f = pl.pallas_call(
        _decode_nms_body,
        out_shape=(
            jax.ShapeDtypeStruct((bsz, 8, _PRE), jnp.float32),
            jax.ShapeDtypeStruct((bsz, _PRE), jnp.float32),
        ),
    )(reg_t, xyz_t, ms_b)

    boxes = boxes8.transpose(0, 2, 1)[..., :7]
    keep_b = keep_f > 0.5
    sel = jnp.argsort((~keep_b).astype(jnp.int32), axis=1, stable=True)
    sel = sel[:, :_POST]
    nkeep = jnp.sum(keep_b, axis=1)
    valid = jnp.arange(_POST)[None, :] < nkeep[:, None]
    out_boxes = jnp.where(valid[..., None],
                          jnp.take_along_axis(boxes, sel[..., None], axis=1),
                          0.0)
    out_scores = jnp.where(valid, jnp.take_along_axis(scores, sel, axis=1),
                           0.0)
    return out_boxes, out_scores


# speedup vs baseline: 82.6289x; 1.2792x over previous
import math

import jax
import jax.numpy as jnp
from jax import lax
from jax.experimental import pallas as pl
from jax.experimental.pallas import tpu as pltpu

_PRE = 2048
_POST = 512
_THRESH = 0.85
_NBIN = 12
_NHEAD = 12
_BIN = 0.5
_SCOPE = 3.0


def _decode_nms_body(reg_ref, xyz_ref, ms_ref, boxes_ref, keep_ref):
    b = reg_ref.shape[0]
    n = reg_ref.shape[2]
    apc = 2.0 * math.pi / _NHEAD

    ms0 = ms_ref[0:1, :]
    ms1 = ms_ref[1:2, :]
    ms2 = ms_ref[2:3, :]

    x1_rows, y1_rows, x2_rows, y2_rows, ar_rows = [], [], [], [], []
    for k in range(b):
        reg = reg_ref[k]
        xyz = xyz_ref[k]
        iot = lax.broadcasted_iota(jnp.int32, (_NBIN, n), 0)

        sx = reg[0:_NBIN, :]
        mx = jnp.max(sx, axis=0, keepdims=True)
        xbin = jnp.min(jnp.where(sx == mx, iot, _NBIN), axis=0, keepdims=True)
        xres = jnp.sum(
            jnp.where(iot == xbin, reg[2 * _NBIN:3 * _NBIN, :], 0.0),
            axis=0, keepdims=True) * _BIN
        posx = xbin.astype(jnp.float32) * _BIN + (_BIN * 0.5 - _SCOPE) + xres
        posx = posx + xyz[0:1, :]

        sz = reg[_NBIN:2 * _NBIN, :]
        mz = jnp.max(sz, axis=0, keepdims=True)
        zbin = jnp.min(jnp.where(sz == mz, iot, _NBIN), axis=0, keepdims=True)
        zres = jnp.sum(
            jnp.where(iot == zbin, reg[3 * _NBIN:4 * _NBIN, :], 0.0),
            axis=0, keepdims=True) * _BIN
        posz = zbin.astype(jnp.float32) * _BIN + (_BIN * 0.5 - _SCOPE) + zres
        posz = posz + xyz[2:3, :]

        posy = xyz[1:2, :] + reg[4 * _NBIN:4 * _NBIN + 1, :]

        hs = 4 * _NBIN + 1
        sr = reg[hs:hs + _NHEAD, :]
        mr = jnp.max(sr, axis=0, keepdims=True)
        rybin = jnp.min(jnp.where(sr == mr, iot, _NHEAD), axis=0, keepdims=True)
        ryres = jnp.sum(
            jnp.where(iot == rybin, reg[hs + _NHEAD:hs + 2 * _NHEAD, :], 0.0),
            axis=0, keepdims=True)
        ry = jnp.mod(rybin.astype(jnp.float32) * apc + ryres * (apc * 0.5),
                     2.0 * math.pi)
        ry = jnp.where(ry > math.pi, ry - 2.0 * math.pi, ry)

        hws = hs + 2 * _NHEAD
        h = reg[hws:hws + 1, :] * ms0 + ms0
        w = reg[hws + 1:hws + 2, :] * ms1 + ms1
        l = reg[hws + 2:hws + 3, :] * ms2 + ms2
        posy = posy + h * 0.5

        boxes_ref[k] = jnp.concatenate(
            [posx, posy, posz, h, w, l, ry, jnp.zeros_like(ry)], axis=0)

        hl = l * 0.5
        hw = w * 0.5
        x1 = posx - hl
        y1 = posz - hw
        x2 = posx + hl
        y2 = posz + hw
        x1_rows.append(x1)
        y1_rows.append(y1)
        x2_rows.append(x2)
        y2_rows.append(y2)
        ar_rows.append((x2 - x1) * (y2 - y1))

    x1m = jnp.concatenate(x1_rows, axis=0)
    y1m = jnp.concatenate(y1_rows, axis=0)
    x2m = jnp.concatenate(x2_rows, axis=0)
    y2m = jnp.concatenate(y2_rows, axis=0)
    arm = jnp.concatenate(ar_rows, axis=0)
    li = lax.broadcasted_iota(jnp.int32, (b, n), 1)

    def cond(carry):
        i, keep_f = carry
        prefix = jnp.sum(keep_f * (li <= i).astype(jnp.float32), axis=1)
        return (i < n) & jnp.any(prefix < float(_POST))

    def body(carry):
        i, keep_f = carry
        sel = li == i
        xi = jnp.sum(jnp.where(sel, x1m, 0.0), axis=1, keepdims=True)
        yi = jnp.sum(jnp.where(sel, y1m, 0.0), axis=1, keepdims=True)
        x2i = jnp.sum(jnp.where(sel, x2m, 0.0), axis=1, keepdims=True)
        y2i = jnp.sum(jnp.where(sel, y2m, 0.0), axis=1, keepdims=True)
        ai = jnp.sum(jnp.where(sel, arm, 0.0), axis=1, keepdims=True)
        ki = jnp.max(jnp.where(sel, keep_f, 0.0), axis=1, keepdims=True)
        xx1 = jnp.maximum(x1m, xi)
        yy1 = jnp.maximum(y1m, yi)
        xx2 = jnp.minimum(x2m, x2i)
        yy2 = jnp.minimum(y2m, y2i)
        inter = jnp.maximum(xx2 - xx1, 0.0) * jnp.maximum(yy2 - yy1, 0.0)
        iou = inter / (ai + arm - inter + 1e-8)
        sup = (iou > _THRESH) & (li > i) & (ki > 0.5)
        return i + 1, jnp.where(sup, 0.0, keep_f)

    _, keep_f = lax.while_loop(
        cond, body, (jnp.int32(0), jnp.ones((b, n), jnp.float32)))
    keep_ref[:] = keep_f


def kernel(rpn_scores, rpn_reg, xyz, mean_size):
    bsz, _ = rpn_scores.shape
    scores, idx = lax.top_k(rpn_scores, _PRE)
    reg_top = jnp.take_along_axis(rpn_reg, idx[:, :, None], axis=1)
    xyz_top = jnp.take_along_axis(xyz, idx[:, :, None], axis=1)
    reg_t = reg_top.transpose(0, 2, 1)
    xyz_t = xyz_top.transpose(0, 2, 1)
    ms_b = jnp.broadcast_to(mean_size[:, None], (3, _PRE))

    boxes8, keep_f = pl.pallas_call(
        _decode_nms_body,
        out_shape=(
            jax.ShapeDtypeStruct((bsz, 8, _PRE), jnp.float32),
            jax.ShapeDtypeStruct((bsz, _PRE), jnp.float32),
        ),
    )(reg_t, xyz_t, ms_b)

    boxes = boxes8.transpose(0, 2, 1)[..., :7]
    keep_b = keep_f > 0.5
    sel = jnp.argsort((~keep_b).astype(jnp.int32), axis=1, stable=True)
    sel = sel[:, :_POST]
    nkeep = jnp.sum(keep_b, axis=1)
    valid = jnp.arange(_POST)[None, :] < nkeep[:, None]
    out_boxes = jnp.where(valid[..., None],
                          jnp.take_along_axis(boxes, sel[..., None], axis=1),
                          0.0)
    out_scores = jnp.where(valid, jnp.take_along_axis(scores, sel, axis=1),
                           0.0)
    return out_boxes, out_scores
